# baseline (speedup 1.0000x reference)
import functools

import jax
import jax.numpy as jnp
from jax import lax
from jax.experimental import pallas as pl
from jax.experimental.pallas import tpu as pltpu
from jax.experimental.pallas import tpu_sc as plsc

NUM_Q = 8
CODEBOOK_SIZE = 1024
DIM = 64
HIDDEN = 1024
BATCH = 16
TLEN = 2048
NTOK = BATCH * TLEN
KDIM = NUM_Q * DIM
NPAIR = KDIM // 128

NC = 2
NS = 16
NW = NC * NS
SLICE_BATCHES = (2, 4, 4, 6)
CHUNK = 128


PDIM = DIM // 2
NQUAD = NUM_Q // 4


def _sc_gather(codes_flat, cb_flat, nbatch, tok_offset):
    ntok_s = nbatch * TLEN
    tok_per_w = ntok_s // NW
    nchunk = tok_per_w // CHUNK
    mesh = plsc.VectorSubcoreMesh(
        core_axis_name="c", subcore_axis_name="s", num_cores=NC, num_subcores=NS
    )

    @functools.partial(
        pl.kernel,
        mesh=mesh,
        out_type=jax.ShapeDtypeStruct((NQUAD, ntok_s, 4 * PDIM), jnp.float32),
        scratch_types=[
            pltpu.VMEM((NUM_Q, CHUNK), jnp.int32),
            pltpu.VMEM((NUM_Q, CHUNK, PDIM), jnp.float32),
            pltpu.SemaphoreType.DMA,
        ],
        compiler_params=pltpu.CompilerParams(use_tc_tiling_on_sc=False),
    )
    def k(codes_hbm, cb_hbm, q_hbm, idx_v, dst_v, sem):
        wid = lax.axis_index("s") * NC + lax.axis_index("c")
        wbase = wid * tok_per_w

        def chunk_body(ci, carry):
            base = wbase + ci * CHUNK
            pltpu.sync_copy(codes_hbm.at[:, pl.ds(base + tok_offset, CHUNK)], idx_v)
            for i in range(1, NUM_Q):
                for j in range(CHUNK // 16):
                    sl = pl.ds(j * 16, 16)
                    idx_v[i, sl] = idx_v[i, sl] + (i * CODEBOOK_SIZE)
            copies = [
                pltpu.async_copy(
                    cb_hbm.at[idx_v.at[i]],
                    dst_v.at[i],
                    sem,
                )
                for i in range(NUM_Q)
            ]
            for cp in copies:
                cp.wait()
            for i in range(NUM_Q):
                pltpu.sync_copy(
                    dst_v.at[i],
                    q_hbm.at[i // 4, pl.ds(base, CHUNK), pl.ds((i % 4) * PDIM, PDIM)],
                )
            return carry

        lax.fori_loop(0, nchunk, chunk_body, 0)

    return k(codes_flat, cb_flat)


TB = 2048


def _tc_matmul_body(q_ref, wt_ref, bt_ref, out_ref):
    u = lax.bitcast_convert_type(q_ref[...], jnp.uint32)
    q_even = lax.bitcast_convert_type(u << 16, jnp.float32).astype(jnp.bfloat16)
    q_odd = lax.bitcast_convert_type(
        u & jnp.uint32(0xFFFF0000), jnp.float32
    ).astype(jnp.bfloat16)
    acc = None
    for par, qp in ((0, q_even), (1, q_odd)):
        qb = jnp.concatenate([qp[g] for g in range(NQUAD)], axis=1)
        wb = jnp.concatenate(
            [wt_ref[par, g] for g in range(NQUAD)], axis=1
        )
        part = lax.dot_general(
            wb, qb,
            dimension_numbers=(((1,), (1,)), ((), ())),
            preferred_element_type=jnp.float32,
        )
        acc = part if acc is None else acc + part
    bsum = jnp.sum(bt_ref[...], axis=1, keepdims=True)
    out_ref[0, :, :] = acc + bsum


def _tc_matmul(carry, q, w_t, b_t, bo, nbatch):
    grid = (nbatch, TLEN // TB)
    body = _tc_matmul_body
    in_specs = [
        pl.BlockSpec(
            (NQUAD, TB, 4 * PDIM), lambda bi, ti: (0, bi * (TLEN // TB) + ti, 0)
        ),
        pl.BlockSpec((2, NQUAD, HIDDEN, 4 * PDIM), lambda bi, ti: (0, 0, 0, 0)),
        pl.BlockSpec((HIDDEN, NUM_Q), lambda bi, ti: (0, 0)),
    ]
    args = (q, w_t, b_t)
    aliases = {}
    if carry is not None:
        body = lambda c_ref, q_ref, wt_ref, bt_ref, out_ref: _tc_matmul_body(
            q_ref, wt_ref, bt_ref, out_ref
        )
        in_specs = [pl.BlockSpec(memory_space=pltpu.MemorySpace.HBM)] + in_specs
        args = (carry,) + args
        aliases = {0: 0}
    return pl.pallas_call(
        body,
        grid=grid,
        in_specs=in_specs,
        out_specs=pl.BlockSpec((1, HIDDEN, TB), lambda bi, ti: (bo + bi, 0, ti)),
        out_shape=jax.ShapeDtypeStruct((BATCH, HIDDEN, TLEN), jnp.float32),
        input_output_aliases=aliases,
        compiler_params=pltpu.CompilerParams(
            dimension_semantics=("arbitrary", "arbitrary"),
        ),
    )(*args)


def kernel(codes, codebooks, W, b):
    codes_flat = codes.astype(jnp.int32).reshape(NUM_Q, NTOK)
    cb_flat = lax.bitcast_convert_type(
        codebooks.reshape(NUM_Q * CODEBOOK_SIZE, PDIM, 2).astype(jnp.bfloat16),
        jnp.float32,
    )
    w_t = (
        W.reshape(2, 4, PDIM, 2, HIDDEN)
        .transpose(3, 0, 4, 1, 2)
        .reshape(2, NQUAD, HIDDEN, 4 * PDIM)
        .astype(jnp.bfloat16)
    )
    b_t = jnp.transpose(b)
    qs = []
    bo = 0
    for nb in SLICE_BATCHES:
        qs.append(_sc_gather(codes_flat, cb_flat, nb, bo * TLEN))
        bo += nb
    w_t, b_t, q0 = lax.optimization_barrier((w_t, b_t, qs[0]))
    qs[0] = q0
    out = None
    bo = 0
    for nb, q in zip(SLICE_BATCHES, qs):
        out = _tc_matmul(out, q, w_t, b_t, bo, nb)
        bo += nb
    return out


# speedup vs baseline: 1.0488x; 1.0488x over previous
import functools

import jax
import jax.numpy as jnp
from jax import lax
from jax.experimental import pallas as pl
from jax.experimental.pallas import tpu as pltpu
from jax.experimental.pallas import tpu_sc as plsc

NUM_Q = 8
CODEBOOK_SIZE = 1024
DIM = 64
HIDDEN = 1024
BATCH = 16
TLEN = 2048
NTOK = BATCH * TLEN
KDIM = NUM_Q * DIM
NPAIR = KDIM // 128

NC = 2
NS = 16
NW = NC * NS
SLICE_BATCHES = (4, 4, 4, 4)
CHUNK = 128


PDIM = DIM // 2
NQUAD = NUM_Q // 4


def _sc_gather(codes_flat, cb_flat, nbatch, tok_offset):
    ntok_s = nbatch * TLEN
    tok_per_w = ntok_s // NW
    nchunk = tok_per_w // CHUNK
    mesh = plsc.VectorSubcoreMesh(
        core_axis_name="c", subcore_axis_name="s", num_cores=NC, num_subcores=NS
    )

    @functools.partial(
        pl.kernel,
        mesh=mesh,
        out_type=jax.ShapeDtypeStruct((NQUAD, ntok_s, 4 * PDIM), jnp.float32),
        scratch_types=[
            pltpu.VMEM((2, NUM_Q, CHUNK), jnp.int32),
            pltpu.VMEM((2, NUM_Q, CHUNK, PDIM), jnp.float32),
            pltpu.SemaphoreType.DMA,
            pltpu.SemaphoreType.DMA,
        ],
        compiler_params=pltpu.CompilerParams(use_tc_tiling_on_sc=False),
    )
    def k(codes_hbm, cb_hbm, q_hbm, idx_v, dst_v, sem0, sem1):
        wid = lax.axis_index("s") * NC + lax.axis_index("c")
        wbase = wid * tok_per_w
        sems = (sem0, sem1)

        def issue(ci, buf):
            base = wbase + ci * CHUNK
            pltpu.sync_copy(
                codes_hbm.at[:, pl.ds(base + tok_offset, CHUNK)], idx_v.at[buf]
            )
            for i in range(1, NUM_Q):
                for j in range(CHUNK // 16):
                    sl = pl.ds(j * 16, 16)
                    idx_v[buf, i, sl] = idx_v[buf, i, sl] + (i * CODEBOOK_SIZE)
            return [
                pltpu.async_copy(
                    cb_hbm.at[idx_v.at[buf, i]],
                    dst_v.at[buf, i],
                    sems[buf],
                )
                for i in range(NUM_Q)
            ]

        def store(ci, buf):
            base = wbase + ci * CHUNK
            for i in range(NUM_Q):
                pltpu.sync_copy(
                    dst_v.at[buf, i],
                    q_hbm.at[i // 4, pl.ds(base, CHUNK), pl.ds((i % 4) * PDIM, PDIM)],
                )

        pending = issue(0, 0)
        for ci in range(1, nchunk):
            nxt = issue(ci, ci % 2)
            for cp in pending:
                cp.wait()
            store(ci - 1, (ci - 1) % 2)
            pending = nxt
        for cp in pending:
            cp.wait()
        store(nchunk - 1, (nchunk - 1) % 2)

    return k(codes_flat, cb_flat)


TB = 2048


def _tc_matmul_body(q_ref, wt_ref, bt_ref, out_ref):
    u = lax.bitcast_convert_type(q_ref[...], jnp.uint32)
    q_even = lax.bitcast_convert_type(u << 16, jnp.float32).astype(jnp.bfloat16)
    q_odd = lax.bitcast_convert_type(
        u & jnp.uint32(0xFFFF0000), jnp.float32
    ).astype(jnp.bfloat16)
    acc = None
    for par, qp in ((0, q_even), (1, q_odd)):
        qb = jnp.concatenate([qp[g] for g in range(NQUAD)], axis=1)
        wb = jnp.concatenate(
            [wt_ref[par, g] for g in range(NQUAD)], axis=1
        )
        part = lax.dot_general(
            wb, qb,
            dimension_numbers=(((1,), (1,)), ((), ())),
            preferred_element_type=jnp.float32,
        )
        acc = part if acc is None else acc + part
    bsum = jnp.sum(bt_ref[...], axis=1, keepdims=True)
    out_ref[0, :, :] = acc + bsum


def _tc_matmul(carry, q, w_t, b_t, bo, nbatch):
    grid = (nbatch, TLEN // TB)
    body = _tc_matmul_body
    in_specs = [
        pl.BlockSpec(
            (NQUAD, TB, 4 * PDIM), lambda bi, ti: (0, bi * (TLEN // TB) + ti, 0)
        ),
        pl.BlockSpec((2, NQUAD, HIDDEN, 4 * PDIM), lambda bi, ti: (0, 0, 0, 0)),
        pl.BlockSpec((HIDDEN, NUM_Q), lambda bi, ti: (0, 0)),
    ]
    args = (q, w_t, b_t)
    aliases = {}
    if carry is not None:
        body = lambda c_ref, q_ref, wt_ref, bt_ref, out_ref: _tc_matmul_body(
            q_ref, wt_ref, bt_ref, out_ref
        )
        in_specs = [pl.BlockSpec(memory_space=pltpu.MemorySpace.HBM)] + in_specs
        args = (carry,) + args
        aliases = {0: 0}
    return pl.pallas_call(
        body,
        grid=grid,
        in_specs=in_specs,
        out_specs=pl.BlockSpec((1, HIDDEN, TB), lambda bi, ti: (bo + bi, 0, ti)),
        out_shape=jax.ShapeDtypeStruct((BATCH, HIDDEN, TLEN), jnp.float32),
        input_output_aliases=aliases,
        compiler_params=pltpu.CompilerParams(
            dimension_semantics=("arbitrary", "arbitrary"),
        ),
    )(*args)


def kernel(codes, codebooks, W, b):
    codes_flat = codes.astype(jnp.int32).reshape(NUM_Q, NTOK)
    cb_flat = lax.bitcast_convert_type(
        codebooks.reshape(NUM_Q * CODEBOOK_SIZE, PDIM, 2).astype(jnp.bfloat16),
        jnp.float32,
    )
    w_t = (
        W.reshape(2, 4, PDIM, 2, HIDDEN)
        .transpose(3, 0, 4, 1, 2)
        .reshape(2, NQUAD, HIDDEN, 4 * PDIM)
        .astype(jnp.bfloat16)
    )
    b_t = jnp.transpose(b)
    qs = []
    bo = 0
    for nb in SLICE_BATCHES:
        qs.append(_sc_gather(codes_flat, cb_flat, nb, bo * TLEN))
        bo += nb
    w_t, b_t, q0 = lax.optimization_barrier((w_t, b_t, qs[0]))
    qs[0] = q0
    out = None
    bo = 0
    for nb, q in zip(SLICE_BATCHES, qs):
        out = _tc_matmul(out, q, w_t, b_t, bo, nb)
        bo += nb
    return out
